# baseline (speedup 1.0000x reference)
import functools
import math

import jax
import jax.numpy as jnp
from jax import lax
from jax.experimental import pallas as pl
from jax.experimental.pallas import tpu as pltpu
from jax.experimental.pallas import tpu_sc as plsc

_ALPHA = 0.1
_THETA = 0.5
_NC = 2
_NS = 16
_LANES = 16
_K = 128


def _num_chunks(E):
    epw = -(-E // (_NC * _NS))
    pch = -(-epw // _K)
    return -(-pch // 16) * 16


@functools.lru_cache(maxsize=None)
def _make_spmm(N, E, H):
    PCH = _num_chunks(E)
    PH = 2
    PPH = PCH // PH
    NBUF = 2
    assert PPH % NBUF == 0
    NG = PPH // NBUF
    NROW = N + 8
    assert N % 8 == 0
    BASE = (N // _NS) // 8 * 8
    EXTRA = N - _NS * BASE
    CW = max(k for k in range(8, min(_K, BASE) + 1, 8) if BASE % k == 0)
    NCW = BASE // CW
    assert EXTRA % 8 == 0 and EXTRA <= _K

    mesh = plsc.VectorSubcoreMesh(core_axis_name="c", subcore_axis_name="s")

    @functools.partial(
        pl.kernel,
        out_type=jax.ShapeDtypeStruct((_NC, N, H), jnp.float32),
        mesh=mesh,
        scratch_types=[
            pltpu.VMEM((PPH, _K), jnp.int32),
            pltpu.VMEM((PPH, _K), jnp.int32),
            pltpu.VMEM((NBUF, _K, H), jnp.float32),
            pltpu.VMEM_SHARED((NROW, H), jnp.float32),
        ] + [pltpu.SemaphoreType.DMA] * (2 * NBUF),
    )
    def spmm(x_hbm, src_hbm, dst_hbm, out_hbm,
             src_v, dst_v, rows_v, acc, *sems):
        gsem = sems[:NBUF]
        ssem = sems[NBUF:]
        c = lax.axis_index("c")
        s = lax.axis_index("s")

        zero16 = jnp.zeros((_LANES,), jnp.float32)

        def _zrow(i, carry):
            for h in range(H // _LANES):
                rows_v[0, i, pl.ds(h * _LANES, _LANES)] = zero16
            return carry

        lax.fori_loop(0, _K, _zrow, 0)

        def _zdst(z):
            return acc.at[pl.ds(s * BASE + z * CW, CW)]

        for z in range(NCW):
            if z >= NBUF:
                pltpu.make_async_copy(rows_v.at[0, pl.ds(0, CW)],
                                      _zdst(z - NBUF),
                                      ssem[z % NBUF]).wait()
            pltpu.async_copy(rows_v.at[0, pl.ds(0, CW)], _zdst(z),
                             ssem[z % NBUF])
        for z in range(max(NCW - NBUF, 0), NCW):
            pltpu.make_async_copy(rows_v.at[0, pl.ds(0, CW)], _zdst(z),
                                  ssem[z % NBUF]).wait()
        if EXTRA:
            @pl.when(s == _NS - 1)
            def _():
                pltpu.sync_copy(rows_v.at[0, pl.ds(0, EXTRA + 8)],
                                acc.at[pl.ds(_NS * BASE, EXTRA + 8)])
        plsc.subcore_barrier()

        def _gather(j, b):
            return pltpu.async_copy(
                x_hbm.at[src_v.at[j]], rows_v.at[b], gsem[b])

        def _gather_wait(j, b):
            pltpu.make_async_copy(
                x_hbm.at[src_v.at[j]], rows_v.at[b], gsem[b]).wait()

        def _scatter(j, b):
            return pltpu.async_copy(
                rows_v.at[b], acc.at[dst_v.at[j]], ssem[b], add=True)

        def _scatter_wait(j, b):
            pltpu.make_async_copy(
                rows_v.at[b], acc.at[dst_v.at[j]], ssem[b]).wait()

        for ph in range(PH):
            c0 = ph * PPH
            cp_s = pltpu.async_copy(
                src_hbm.at[c, s, pl.ds(c0, PPH)], src_v, gsem[0])
            cp_d = pltpu.async_copy(
                dst_hbm.at[c, s, pl.ds(c0, PPH)], dst_v, gsem[1])
            cp_s.wait()
            cp_d.wait()

            for b in range(NBUF):
                _gather(b, b)

            def _grp(g, carry):
                j0 = g * NBUF
                for b in range(NBUF):
                    _gather_wait(j0 + b, b)
                    _scatter(j0 + b, b)
                for b in range(NBUF):
                    _scatter_wait(j0 + b, b)
                    _gather(j0 + NBUF + b, b)
                return carry

            lax.fori_loop(0, NG - 1, _grp, 0)
            j0 = (NG - 1) * NBUF
            for b in range(NBUF):
                _gather_wait(j0 + b, b)
                _scatter(j0 + b, b)
            for b in range(NBUF):
                _scatter_wait(j0 + b, b)
        plsc.subcore_barrier()

        def _wdst(z):
            return out_hbm.at[c, pl.ds(s * BASE + z * CW, CW)]

        for z in range(NCW):
            b = z % NBUF
            if z >= NBUF:
                pltpu.make_async_copy(rows_v.at[b, pl.ds(0, CW)],
                                      _wdst(z - NBUF), ssem[b]).wait()
            pltpu.sync_copy(acc.at[pl.ds(s * BASE + z * CW, CW)],
                            rows_v.at[b, pl.ds(0, CW)])
            pltpu.async_copy(rows_v.at[b, pl.ds(0, CW)], _wdst(z), ssem[b])
        for z in range(max(NCW - NBUF, 0), NCW):
            pltpu.make_async_copy(rows_v.at[z % NBUF, pl.ds(0, CW)],
                                  _wdst(z), ssem[z % NBUF]).wait()
        if EXTRA:
            @pl.when(s == _NS - 1)
            def _():
                r0 = _NS * BASE
                pltpu.sync_copy(acc.at[pl.ds(r0, EXTRA)],
                                rows_v.at[0, pl.ds(0, EXTRA)])
                pltpu.sync_copy(rows_v.at[0, pl.ds(0, EXTRA)],
                                out_hbm.at[c, pl.ds(r0, EXTRA)])

    return spmm


def _row_block(N):
    for br in (2000, 1000, 500, 250, 200, 125, 100, 50, 25, 8, 5, 4, 2, 1):
        if N % br == 0 and br % 8 == 0:
            return br
    return N


def _dot(a, b):
    return jax.lax.dot_general(
        a, b, (((1,), (0,)), ((), ())),
        precision=jax.lax.Precision.HIGHEST,
        preferred_element_type=jnp.float32)


@functools.lru_cache(maxsize=None)
def _make_lin0(N, F, H):
    BR = _row_block(N)

    def body(x_ref, w_ref, b_ref, o_ref):
        o_ref[...] = jnp.maximum(
            _dot(x_ref[...], w_ref[...]) + b_ref[...], 0.0)

    return pl.pallas_call(
        body,
        out_shape=jax.ShapeDtypeStruct((N, H), jnp.float32),
        grid=(N // BR,),
        in_specs=[
            pl.BlockSpec((BR, F), lambda i: (i, 0)),
            pl.BlockSpec((F, H), lambda i: (0, 0)),
            pl.BlockSpec((1, H), lambda i: (0, 0)),
        ],
        out_specs=pl.BlockSpec((BR, H), lambda i: (i, 0)),
    )


@functools.lru_cache(maxsize=None)
def _make_layer(N, H, beta, last, C=0):
    BR = _row_block(N)

    def body(*refs):
        if last:
            p_ref, x0_ref, xc_ref, w_ref, w1_ref, b1_ref, o_ref = refs
        else:
            p_ref, x0_ref, xc_ref, w_ref, o_ref = refs
        agg = p_ref[0] + p_ref[1]
        hmix = (1.0 - _ALPHA) * agg + _ALPHA * x0_ref[...]
        conv = (1.0 - beta) * hmix + beta * _dot(hmix, w_ref[...])
        xnew = jnp.maximum(conv + xc_ref[...], 0.0)
        if last:
            o_ref[...] = _dot(xnew, w1_ref[...]) + b1_ref[...]
        else:
            o_ref[...] = xnew

    in_specs = [
        pl.BlockSpec((_NC, BR, H), lambda i: (0, i, 0)),
        pl.BlockSpec((BR, H), lambda i: (i, 0)),
        pl.BlockSpec((BR, H), lambda i: (i, 0)),
        pl.BlockSpec((H, H), lambda i: (0, 0)),
    ]
    if last:
        in_specs += [
            pl.BlockSpec((H, C), lambda i: (0, 0)),
            pl.BlockSpec((1, C), lambda i: (0, 0)),
        ]
        out_dim = C
    else:
        out_dim = H

    return pl.pallas_call(
        body,
        out_shape=jax.ShapeDtypeStruct((N, out_dim), jnp.float32),
        grid=(N // BR,),
        in_specs=in_specs,
        out_specs=pl.BlockSpec((BR, out_dim), lambda i: (i, 0)),
    )


def kernel(x, edge_index, lin0_w, lin0_b, convW, lin1_w, lin1_b):
    N, F = x.shape
    H = lin0_w.shape[1]
    C = lin1_w.shape[1]
    L = convW.shape[0]
    E = edge_index.shape[1]

    NW = _NC * _NS
    PCH = _num_chunks(E)
    EP = NW * PCH * _K
    src_p = jnp.concatenate(
        [edge_index[0], jnp.zeros((EP - E,), jnp.int32)]
    ).reshape(_NC, _NS, PCH, _K)
    dst_p = jnp.concatenate(
        [edge_index[1], jnp.full((EP - E,), N, jnp.int32)]
    ).reshape(_NC, _NS, PCH, _K)

    spmm = _make_spmm(N, E, H)
    xc = _make_lin0(N, F, H)(x, lin0_w, lin0_b.reshape(1, H))
    x0 = xc
    for layer in range(L):
        beta = float(math.log(_THETA / (layer + 1) + 1.0))
        parts = spmm(xc, src_p, dst_p)
        if layer < L - 1:
            xc = _make_layer(N, H, beta, False)(parts, x0, xc, convW[layer])
        else:
            out = _make_layer(N, H, beta, True, C)(
                parts, x0, xc, convW[layer], lin1_w, lin1_b.reshape(1, C))
    return out


# speedup vs baseline: 3.1828x; 3.1828x over previous
import functools
import math

import jax
import jax.numpy as jnp
from jax import lax
from jax.experimental import pallas as pl
from jax.experimental.pallas import tpu as pltpu
from jax.experimental.pallas import tpu_sc as plsc

_ALPHA = 0.1
_THETA = 0.5
_NC = 2
_NS = 16
_LANES = 16


@functools.lru_cache(maxsize=None)
def _make_spmm(N, E, H):
    Hh = H // _NC
    assert E % _NS == 0
    EPW = E // _NS
    K = max(k for k in range(1, 129) if EPW % k == 0)
    NCHUNK = EPW // K
    NBUF = 8
    PH = 2
    assert NCHUNK % (PH * NBUF) == 0
    NHALF = NCHUNK // PH
    NG = NHALF // NBUF
    BASE = (N // _NS) // 8 * 8
    EXTRA = N - _NS * BASE
    CW = max(k for k in range(8, min(K, BASE) + 1, 8) if BASE % k == 0)
    NCW = BASE // CW
    assert EXTRA % 8 == 0 and EXTRA <= K

    mesh = plsc.VectorSubcoreMesh(core_axis_name="c", subcore_axis_name="s")

    @functools.partial(
        pl.kernel,
        out_type=jax.ShapeDtypeStruct((_NC, N, Hh), jnp.float32),
        mesh=mesh,
        scratch_types=[
            pltpu.VMEM((NHALF, K), jnp.int32),
            pltpu.VMEM((NHALF, K), jnp.int32),
            pltpu.VMEM((NBUF, K, Hh), jnp.float32),
            pltpu.VMEM_SHARED((N, Hh), jnp.float32),
        ] + [pltpu.SemaphoreType.DMA] * (2 * NBUF),
        compiler_params=pltpu.CompilerParams(use_tc_tiling_on_sc=False),
    )
    def spmm(x2_hbm, src_hbm, dst_hbm, out_hbm,
             src_v, dst_v, rows_v, acc, *sems):
        gsem = sems[:NBUF]
        ssem = sems[NBUF:]
        c = lax.axis_index("c")
        s = lax.axis_index("s")

        zero16 = jnp.zeros((_LANES,), jnp.float32)

        def _zrow(i, carry):
            for h in range(Hh // _LANES):
                rows_v[0, i, pl.ds(h * _LANES, _LANES)] = zero16
            return carry

        lax.fori_loop(0, K, _zrow, 0)
        for z in range(NCW):
            pltpu.async_copy(rows_v.at[0, pl.ds(0, CW)],
                             acc.at[pl.ds(s * BASE + z * CW, CW)], ssem[z])
        if EXTRA:
            @pl.when(s == _NS - 1)
            def _():
                pltpu.async_copy(rows_v.at[0, pl.ds(0, EXTRA)],
                                 acc.at[pl.ds(_NS * BASE, EXTRA)],
                                 ssem[NCW])
        for z in range(NCW):
            pltpu.make_async_copy(rows_v.at[0, pl.ds(0, CW)],
                                  acc.at[pl.ds(s * BASE + z * CW, CW)],
                                  ssem[z]).wait()
        if EXTRA:
            @pl.when(s == _NS - 1)
            def _():
                pltpu.make_async_copy(rows_v.at[0, pl.ds(0, EXTRA)],
                                      acc.at[pl.ds(_NS * BASE, EXTRA)],
                                      ssem[NCW]).wait()
        plsc.subcore_barrier()

        def _gather(j, b):
            return pltpu.async_copy(
                x2_hbm.at[src_v.at[j]], rows_v.at[b], gsem[b])

        def _gather_wait(j, b):
            pltpu.make_async_copy(
                x2_hbm.at[src_v.at[j]], rows_v.at[b], gsem[b]).wait()

        def _scatter(j, b):
            return pltpu.async_copy(
                rows_v.at[b], acc.at[dst_v.at[j]], ssem[b], add=True)

        def _scatter_wait(j, b):
            pltpu.make_async_copy(
                rows_v.at[b], acc.at[dst_v.at[j]], ssem[b]).wait()

        for ph in range(PH):
            c0 = ph * NHALF
            cp_s = pltpu.async_copy(
                src_hbm.at[c, s, pl.ds(c0, NHALF)], src_v, gsem[0])
            cp_d = pltpu.async_copy(
                dst_hbm.at[s, pl.ds(c0, NHALF)], dst_v, gsem[1])
            cp_s.wait()
            cp_d.wait()

            for b in range(NBUF):
                _gather(b, b)

            def _grp(g, carry):
                j0 = g * NBUF
                for b in range(NBUF):
                    _gather_wait(j0 + b, b)
                    _scatter(j0 + b, b)
                for b in range(NBUF):
                    _scatter_wait(j0 + b, b)
                    _gather(j0 + NBUF + b, b)
                return carry

            lax.fori_loop(0, NG - 1, _grp, 0)
            j0 = (NG - 1) * NBUF
            for b in range(NBUF):
                _gather_wait(j0 + b, b)
                _scatter(j0 + b, b)
            for b in range(NBUF):
                _scatter_wait(j0 + b, b)
        plsc.subcore_barrier()

        for z in range(NCW):
            r0 = s * BASE + z * CW
            pltpu.async_copy(acc.at[pl.ds(r0, CW)],
                             rows_v.at[z, pl.ds(0, CW)], gsem[z])
        for z in range(NCW):
            r0 = s * BASE + z * CW
            pltpu.make_async_copy(acc.at[pl.ds(r0, CW)],
                                  rows_v.at[z, pl.ds(0, CW)],
                                  gsem[z]).wait()
            pltpu.async_copy(rows_v.at[z, pl.ds(0, CW)],
                             out_hbm.at[c, pl.ds(r0, CW)], ssem[z])
        if EXTRA:
            @pl.when(s == _NS - 1)
            def _():
                r0 = _NS * BASE
                pltpu.sync_copy(acc.at[pl.ds(r0, EXTRA)],
                                rows_v.at[NCW, pl.ds(0, EXTRA)])
                pltpu.async_copy(rows_v.at[NCW, pl.ds(0, EXTRA)],
                                 out_hbm.at[c, pl.ds(r0, EXTRA)],
                                 ssem[NCW])
        for z in range(NCW):
            r0 = s * BASE + z * CW
            pltpu.make_async_copy(rows_v.at[z, pl.ds(0, CW)],
                                  out_hbm.at[c, pl.ds(r0, CW)],
                                  ssem[z]).wait()
        if EXTRA:
            @pl.when(s == _NS - 1)
            def _():
                pltpu.make_async_copy(
                    rows_v.at[NCW, pl.ds(0, EXTRA)],
                    out_hbm.at[c, pl.ds(_NS * BASE, EXTRA)],
                    ssem[NCW]).wait()

    return spmm


def _row_block(N):
    for br in (2000, 1000, 500, 250, 200, 125, 100, 50, 25, 8, 5, 4, 2, 1):
        if N % br == 0 and br % 8 == 0:
            return br
    return N


def _dot(a, b):
    return jax.lax.dot_general(
        a, b, (((1,), (0,)), ((), ())),
        precision=jax.lax.Precision.HIGHEST,
        preferred_element_type=jnp.float32)


@functools.lru_cache(maxsize=None)
def _make_lin0(N, F, H):
    BR = _row_block(N)

    Hh = H // _NC

    def body(x_ref, w_ref, b_ref, o_ref):
        h = jnp.maximum(_dot(x_ref[...], w_ref[...]) + b_ref[...], 0.0)
        o_ref[0] = h[:, :Hh]
        o_ref[1] = h[:, Hh:]

    return pl.pallas_call(
        body,
        out_shape=jax.ShapeDtypeStruct((_NC, N, Hh), jnp.float32),
        grid=(N // BR,),
        in_specs=[
            pl.BlockSpec((BR, F), lambda i: (i, 0)),
            pl.BlockSpec((F, H), lambda i: (0, 0)),
            pl.BlockSpec((1, H), lambda i: (0, 0)),
        ],
        out_specs=pl.BlockSpec((_NC, BR, Hh), lambda i: (0, i, 0)),
    )


@functools.lru_cache(maxsize=None)
def _make_layer(N, H, beta, last, C=0):
    BR = _row_block(N)
    Hh = H // _NC

    def body(*refs):
        if last:
            p_ref, x0_ref, xc_ref, w_ref, w1_ref, b1_ref, o_ref = refs
        else:
            p_ref, x0_ref, xc_ref, w_ref, o_ref = refs
        hm0 = (1.0 - _ALPHA) * p_ref[0] + _ALPHA * x0_ref[0]
        hm1 = (1.0 - _ALPHA) * p_ref[1] + _ALPHA * x0_ref[1]
        hmix = jnp.concatenate([hm0, hm1], axis=1)
        conv = (1.0 - beta) * hmix + beta * _dot(hmix, w_ref[...])
        xc = jnp.concatenate([xc_ref[0], xc_ref[1]], axis=1)
        xnew = jnp.maximum(conv + xc, 0.0)
        if last:
            o_ref[...] = _dot(xnew, w1_ref[...]) + b1_ref[...]
        else:
            o_ref[0] = xnew[:, :Hh]
            o_ref[1] = xnew[:, Hh:]

    in_specs = [
        pl.BlockSpec((_NC, BR, Hh), lambda i: (0, i, 0)),
        pl.BlockSpec((_NC, BR, Hh), lambda i: (0, i, 0)),
        pl.BlockSpec((_NC, BR, Hh), lambda i: (0, i, 0)),
        pl.BlockSpec((H, H), lambda i: (0, 0)),
    ]
    if last:
        in_specs += [
            pl.BlockSpec((H, C), lambda i: (0, 0)),
            pl.BlockSpec((1, C), lambda i: (0, 0)),
        ]
        out_shape = jax.ShapeDtypeStruct((N, C), jnp.float32)
        out_specs = pl.BlockSpec((BR, C), lambda i: (i, 0))
    else:
        out_shape = jax.ShapeDtypeStruct((_NC, N, Hh), jnp.float32)
        out_specs = pl.BlockSpec((_NC, BR, Hh), lambda i: (0, i, 0))

    return pl.pallas_call(
        body,
        out_shape=out_shape,
        grid=(N // BR,),
        in_specs=in_specs,
        out_specs=out_specs,
    )


def kernel(x, edge_index, lin0_w, lin0_b, convW, lin1_w, lin1_b):
    N, F = x.shape
    H = lin0_w.shape[1]
    C = lin1_w.shape[1]
    L = convW.shape[0]
    E = edge_index.shape[1]

    EPW = E // _NS
    K = max(k for k in range(1, 129) if EPW % k == 0)
    src2 = (edge_index[0].reshape(1, _NS, EPW // K, K)
            + (jnp.arange(_NC, dtype=jnp.int32) * N).reshape(_NC, 1, 1, 1))
    dst2 = edge_index[1].reshape(_NS, EPW // K, K)

    spmm = _make_spmm(N, E, H)
    xc2 = _make_lin0(N, F, H)(x, lin0_w, lin0_b.reshape(1, H))
    x02 = xc2
    for layer in range(L):
        beta = float(math.log(_THETA / (layer + 1) + 1.0))
        parts = spmm(xc2.reshape(_NC * N, H // _NC), src2, dst2)
        if layer < L - 1:
            xc2 = _make_layer(N, H, beta, False)(parts, x02, xc2, convW[layer])
        else:
            out = _make_layer(N, H, beta, True, C)(
                parts, x02, xc2, convW[layer], lin1_w, lin1_b.reshape(1, C))
    return out


# speedup vs baseline: 3.4494x; 1.0837x over previous
import functools
import math

import jax
import jax.numpy as jnp
from jax import lax
from jax.experimental import pallas as pl
from jax.experimental.pallas import tpu as pltpu
from jax.experimental.pallas import tpu_sc as plsc

_ALPHA = 0.1
_THETA = 0.5
_NC = 2
_NS = 16
_LANES = 16


@functools.lru_cache(maxsize=None)
def _make_spmm(N, E, H):
    Hh = H // _NC
    assert E % _NS == 0
    EPW = E // _NS
    K = max(k for k in range(1, 129) if EPW % k == 0)
    NCHUNK = EPW // K
    NBUF = 8
    PH = 2
    assert NCHUNK % (PH * NBUF) == 0
    NHALF = NCHUNK // PH
    NG = NHALF // NBUF
    BASE = (N // _NS) // 8 * 8
    EXTRA = N - _NS * BASE
    CW = max(k for k in range(8, min(K, BASE) + 1, 8) if BASE % k == 0)
    NCW = BASE // CW
    assert EXTRA % 8 == 0 and EXTRA <= K

    mesh = plsc.VectorSubcoreMesh(core_axis_name="c", subcore_axis_name="s")

    @functools.partial(
        pl.kernel,
        out_type=jax.ShapeDtypeStruct((_NC, N, Hh), jnp.float32),
        mesh=mesh,
        scratch_types=[
            pltpu.VMEM((NHALF, K), jnp.int32),
            pltpu.VMEM((NHALF, K), jnp.int32),
            pltpu.VMEM((NBUF, K, Hh), jnp.float32),
            pltpu.VMEM_SHARED((N, Hh), jnp.float32),
        ] + [pltpu.SemaphoreType.DMA] * (2 * NBUF),
        compiler_params=pltpu.CompilerParams(use_tc_tiling_on_sc=False),
    )
    def spmm(x2_hbm, src_hbm, dst_hbm, out_hbm,
             src_v, dst_v, rows_v, acc, *sems):
        gsem = sems[:NBUF]
        ssem = sems[NBUF:]
        c = lax.axis_index("c")
        s = lax.axis_index("s")

        zero16 = jnp.zeros((_LANES,), jnp.float32)

        def _zrow(i, carry):
            for h in range(Hh // _LANES):
                rows_v[0, i, pl.ds(h * _LANES, _LANES)] = zero16
            return carry

        lax.fori_loop(0, K, _zrow, 0)
        for z in range(NCW):
            pltpu.async_copy(rows_v.at[0, pl.ds(0, CW)],
                             acc.at[pl.ds(s * BASE + z * CW, CW)], ssem[z])
        if EXTRA:
            @pl.when(s == _NS - 1)
            def _():
                pltpu.async_copy(rows_v.at[0, pl.ds(0, EXTRA)],
                                 acc.at[pl.ds(_NS * BASE, EXTRA)],
                                 ssem[NCW])
        for z in range(NCW):
            pltpu.make_async_copy(rows_v.at[0, pl.ds(0, CW)],
                                  acc.at[pl.ds(s * BASE + z * CW, CW)],
                                  ssem[z]).wait()
        if EXTRA:
            @pl.when(s == _NS - 1)
            def _():
                pltpu.make_async_copy(rows_v.at[0, pl.ds(0, EXTRA)],
                                      acc.at[pl.ds(_NS * BASE, EXTRA)],
                                      ssem[NCW]).wait()
        plsc.subcore_barrier()

        def _gather(j, b):
            return pltpu.async_copy(
                x2_hbm.at[src_v.at[j]], rows_v.at[b], gsem[b])

        def _gather_wait(j, b):
            pltpu.make_async_copy(
                x2_hbm.at[src_v.at[j]], rows_v.at[b], gsem[b]).wait()

        def _scatter(j, b):
            return pltpu.async_copy(
                rows_v.at[b], acc.at[dst_v.at[j]], ssem[b], add=True)

        def _scatter_wait(j, b):
            pltpu.make_async_copy(
                rows_v.at[b], acc.at[dst_v.at[j]], ssem[b]).wait()

        for ph in range(PH):
            c0 = ph * NHALF
            cp_s = pltpu.async_copy(
                src_hbm.at[c, s, pl.ds(c0, NHALF)], src_v, gsem[0])
            cp_d = pltpu.async_copy(
                dst_hbm.at[s, pl.ds(c0, NHALF)], dst_v, gsem[1])
            cp_s.wait()
            cp_d.wait()

            for b in range(NBUF):
                _gather(b, b)

            def _grp(g, carry):
                j0 = g * NBUF
                for b in range(NBUF):
                    _gather_wait(j0 + b, b)
                    _scatter(j0 + b, b)
                for b in range(NBUF):
                    _scatter_wait(j0 + b, b)
                    _gather(j0 + NBUF + b, b)
                return carry

            lax.fori_loop(0, NG - 1, _grp, 0)
            j0 = (NG - 1) * NBUF
            for b in range(NBUF):
                _gather_wait(j0 + b, b)
                _scatter(j0 + b, b)
            for b in range(NBUF):
                _scatter_wait(j0 + b, b)
        plsc.subcore_barrier()

        for z in range(NCW):
            r0 = s * BASE + z * CW
            pltpu.async_copy(acc.at[pl.ds(r0, CW)],
                             rows_v.at[z, pl.ds(0, CW)], gsem[z])
        for z in range(NCW):
            r0 = s * BASE + z * CW
            pltpu.make_async_copy(acc.at[pl.ds(r0, CW)],
                                  rows_v.at[z, pl.ds(0, CW)],
                                  gsem[z]).wait()
            pltpu.async_copy(rows_v.at[z, pl.ds(0, CW)],
                             out_hbm.at[c, pl.ds(r0, CW)], ssem[z])
        if EXTRA:
            @pl.when(s == _NS - 1)
            def _():
                r0 = _NS * BASE
                pltpu.sync_copy(acc.at[pl.ds(r0, EXTRA)],
                                rows_v.at[NCW, pl.ds(0, EXTRA)])
                pltpu.async_copy(rows_v.at[NCW, pl.ds(0, EXTRA)],
                                 out_hbm.at[c, pl.ds(r0, EXTRA)],
                                 ssem[NCW])
        for z in range(NCW):
            r0 = s * BASE + z * CW
            pltpu.make_async_copy(rows_v.at[z, pl.ds(0, CW)],
                                  out_hbm.at[c, pl.ds(r0, CW)],
                                  ssem[z]).wait()
        if EXTRA:
            @pl.when(s == _NS - 1)
            def _():
                pltpu.make_async_copy(
                    rows_v.at[NCW, pl.ds(0, EXTRA)],
                    out_hbm.at[c, pl.ds(_NS * BASE, EXTRA)],
                    ssem[NCW]).wait()

    return spmm


def _row_block(N):
    for br in (2000, 1000, 500, 250, 200, 125, 100, 50, 25, 8, 5, 4, 2, 1):
        if N % br == 0 and br % 8 == 0:
            return br
    return N


def _dot(a, b):
    return jax.lax.dot_general(
        a, b, (((1,), (0,)), ((), ())),
        precision=jax.lax.Precision.HIGHEST,
        preferred_element_type=jnp.float32)


@functools.lru_cache(maxsize=None)
def _make_lin0(N, F, H):
    BR = _row_block(N)

    def body(x_ref, w_ref, b_ref, o_ref):
        o_ref[...] = jnp.maximum(
            _dot(x_ref[...], w_ref[...]) + b_ref[...], 0.0)

    return pl.pallas_call(
        body,
        out_shape=jax.ShapeDtypeStruct((N, H), jnp.float32),
        grid=(N // BR,),
        in_specs=[
            pl.BlockSpec((BR, F), lambda i: (i, 0)),
            pl.BlockSpec((F, H), lambda i: (0, 0)),
            pl.BlockSpec((1, H), lambda i: (0, 0)),
        ],
        out_specs=pl.BlockSpec((BR, H), lambda i: (i, 0)),
    )


@functools.lru_cache(maxsize=None)
def _make_layer(N, H, beta, last, C=0):
    BR = _row_block(N)
    Hh = H // _NC

    def body(*refs):
        if last:
            p_ref, x0_ref, xc_ref, w_ref, w1_ref, b1_ref, o_ref = refs
        else:
            p_ref, x0_ref, xc_ref, w_ref, o_ref = refs
        agg = jnp.concatenate([p_ref[0], p_ref[1]], axis=1)
        hmix = (1.0 - _ALPHA) * agg + _ALPHA * x0_ref[...]
        conv = (1.0 - beta) * hmix + beta * _dot(hmix, w_ref[...])
        xnew = jnp.maximum(conv + xc_ref[...], 0.0)
        if last:
            o_ref[...] = _dot(xnew, w1_ref[...]) + b1_ref[...]
        else:
            o_ref[...] = xnew

    in_specs = [
        pl.BlockSpec((_NC, BR, Hh), lambda i: (0, i, 0)),
        pl.BlockSpec((BR, H), lambda i: (i, 0)),
        pl.BlockSpec((BR, H), lambda i: (i, 0)),
        pl.BlockSpec((H, H), lambda i: (0, 0)),
    ]
    if last:
        in_specs += [
            pl.BlockSpec((H, C), lambda i: (0, 0)),
            pl.BlockSpec((1, C), lambda i: (0, 0)),
        ]
        out_dim = C
    else:
        out_dim = H

    return pl.pallas_call(
        body,
        out_shape=jax.ShapeDtypeStruct((N, out_dim), jnp.float32),
        grid=(N // BR,),
        in_specs=in_specs,
        out_specs=pl.BlockSpec((BR, out_dim), lambda i: (i, 0)),
    )


def kernel(x, edge_index, lin0_w, lin0_b, convW, lin1_w, lin1_b):
    N, F = x.shape
    H = lin0_w.shape[1]
    C = lin1_w.shape[1]
    L = convW.shape[0]
    E = edge_index.shape[1]

    EPW = E // _NS
    K = max(k for k in range(1, 129) if EPW % k == 0)
    src2 = (2 * edge_index[0].reshape(1, _NS, EPW // K, K)
            + jnp.arange(_NC, dtype=jnp.int32).reshape(_NC, 1, 1, 1))
    dst2 = edge_index[1].reshape(_NS, EPW // K, K)

    spmm = _make_spmm(N, E, H)
    xc = _make_lin0(N, F, H)(x, lin0_w, lin0_b.reshape(1, H))
    x0 = xc
    for layer in range(L):
        beta = float(math.log(_THETA / (layer + 1) + 1.0))
        parts = spmm(xc.reshape(_NC * N, H // _NC), src2, dst2)
        if layer < L - 1:
            xc = _make_layer(N, H, beta, False)(parts, x0, xc, convW[layer])
        else:
            out = _make_layer(N, H, beta, True, C)(
                parts, x0, xc, convW[layer], lin1_w, lin1_b.reshape(1, C))
    return out
